# Initial kernel scaffold; baseline (speedup 1.0000x reference)
import jax
import jax.numpy as jnp
from jax.experimental import pallas as pl


def kernel(x, edge_index, edge_attr, u, batch, W1, b1, W2, b2):
    raise NotImplementedError("write your pallas kernel here")


# speedup vs baseline: 2.0051x; 2.0051x over previous
import functools

import jax
import jax.numpy as jnp
from jax import lax
from jax.experimental import pallas as pl
from jax.experimental.pallas import tpu as pltpu
from jax.experimental.pallas import tpu_sc as plsc

N = 100000
E = 3200000
DX = 128
DE = 16
DU = 64
B = 128
H = 64

NC = 2
NS = 16
NW = NC * NS

RB = 100
ROWS = E // RB
ROWS_W = ROWS // NW
KC = 8
ITERS = ROWS_W // KC
NPAD = 100096
NPT = NPAD // NS


def _scatter_body(src_hbm, attr_hbm, zs_hbm, ones_hbm,
                  sums_out, cnt_out, acc_sh, idx_v, attr_v, ones_v):
  core = lax.axis_index("c")
  sub = lax.axis_index("s")
  w = sub * NC + core
  row_lo = w * ROWS_W

  pltpu.sync_copy(ones_hbm, ones_v)
  pltpu.sync_copy(zs_hbm.at[pl.ds(sub * NPT, NPT)],
                  acc_sh.at[pl.ds(sub * NPT, NPT)])
  plsc.subcore_barrier()

  def attr_chunk(i, carry):
    row0 = row_lo + i * KC
    pltpu.sync_copy(src_hbm.at[pl.ds(row0, KC)], idx_v)
    pltpu.sync_copy(attr_hbm.at[pl.ds(row0, KC)], attr_v)
    for j in range(KC):
      pltpu.sync_copy(attr_v.at[j], acc_sh.at[idx_v.at[j]], add=True)
    return carry

  lax.fori_loop(0, ITERS, attr_chunk, 0)
  plsc.subcore_barrier()

  pltpu.sync_copy(acc_sh.at[pl.ds(sub * NPT, NPT)],
                  sums_out.at[core, pl.ds(sub * NPT, NPT)])
  plsc.subcore_barrier()

  pltpu.sync_copy(zs_hbm.at[pl.ds(sub * NPT, NPT)],
                  acc_sh.at[pl.ds(sub * NPT, NPT)])
  plsc.subcore_barrier()

  def cnt_chunk(i, carry):
    row0 = row_lo + i * KC
    pltpu.sync_copy(src_hbm.at[pl.ds(row0, KC)], idx_v)
    for j in range(KC):
      pltpu.sync_copy(ones_v, acc_sh.at[idx_v.at[j]], add=True)
    return carry

  lax.fori_loop(0, ITERS, cnt_chunk, 0)
  plsc.subcore_barrier()

  pltpu.sync_copy(acc_sh.at[pl.ds(sub * NPT, NPT)],
                  cnt_out.at[core, pl.ds(sub * NPT, NPT)])


@functools.cache
def _scatter():
  return pl.kernel(
      _scatter_body,
      out_type=[
          jax.ShapeDtypeStruct((NC, NPAD, DE), jnp.float32),
          jax.ShapeDtypeStruct((NC, NPAD, DE), jnp.float32),
      ],
      mesh=plsc.VectorSubcoreMesh(core_axis_name="c", subcore_axis_name="s",
                                  num_cores=NC, num_subcores=NS),
      compiler_params=pltpu.CompilerParams(use_tc_tiling_on_sc=False),
      scratch_types=[
          pltpu.VMEM_SHARED((NPAD, DE), jnp.float32),
          pltpu.VMEM((KC, RB), jnp.int32),
          pltpu.VMEM((KC, RB, DE), jnp.float32),
          pltpu.VMEM((RB, DE), jnp.float32),
      ],
  )


BN = 1000
GRID = N // BN


def _mlp_body(x_ref, s_ref, c_ref, b_ref, u_ref, w1x_ref, w1e_ref, w1u_ref,
              w2_ref, b1_ref, b2_ref, o_ref):
  f32 = jnp.float32
  hi = jax.lax.Precision.HIGHEST
  s = s_ref[0] + s_ref[1]
  c = c_ref[0, :, 0:1] + c_ref[1, :, 0:1]
  ve = s / jnp.maximum(c, 1.0)
  oh = (b_ref[...] == lax.broadcasted_iota(jnp.int32, (1, B), 1)).astype(f32)
  uproj = jnp.dot(u_ref[...], w1u_ref[...], precision=hi,
                  preferred_element_type=f32)
  acc = (jnp.dot(x_ref[...], w1x_ref[...], precision=hi,
                 preferred_element_type=f32)
         + jnp.dot(ve, w1e_ref[...], precision=hi, preferred_element_type=f32)
         + jnp.dot(oh, uproj, precision=hi, preferred_element_type=f32)
         + b1_ref[...])
  h = jnp.maximum(acc, 0.0)
  o_ref[...] = jnp.maximum(
      jnp.dot(h, w2_ref[...], precision=hi, preferred_element_type=f32)
      + b2_ref[...], 0.0)


def _mlp(x, sums, cnts, batch2, u, w1x, w1e, w1u, w2t, b1r, b2r):
  full = lambda shape: pl.BlockSpec(shape, lambda i: tuple(0 for _ in shape))
  return pl.pallas_call(
      _mlp_body,
      grid=(GRID,),
      in_specs=[
          pl.BlockSpec((BN, DX), lambda i: (i, 0)),
          pl.BlockSpec((NC, BN, DE), lambda i: (0, i, 0)),
          pl.BlockSpec((NC, BN, DE), lambda i: (0, i, 0)),
          pl.BlockSpec((BN, 1), lambda i: (i, 0)),
          full((B, DU)),
          full((DX, H)),
          full((DE, H)),
          full((DU, H)),
          full((H, H)),
          full((1, H)),
          full((1, H)),
      ],
      out_specs=pl.BlockSpec((BN, H), lambda i: (i, 0)),
      out_shape=jax.ShapeDtypeStruct((N, H), jnp.float32),
  )(x, sums, cnts, batch2, u, w1x, w1e, w1u, w2t, b1r, b2r)


def kernel(x, edge_index, edge_attr, u, batch, W1, b1, W2, b2):
  src2 = edge_index[0].reshape(ROWS, RB)
  attr3 = edge_attr.reshape(ROWS, RB, DE)
  zs = jnp.zeros((NPAD, DE), jnp.float32)
  ones = jnp.zeros((RB, DE), jnp.float32).at[:, 0].set(1.0)
  sums, cnts = _scatter()(src2, attr3, zs, ones)

  W1T = W1.T
  return _mlp(x, sums, cnts, batch.reshape(N, 1), u,
              W1T[:DX], W1T[DX:DX + DE], W1T[DX + DE:], W2.T,
              b1.reshape(1, H), b2.reshape(1, H))


# speedup vs baseline: 5.2999x; 2.6432x over previous
import functools

import jax
import jax.numpy as jnp
from jax import lax
from jax.experimental import pallas as pl
from jax.experimental.pallas import tpu as pltpu
from jax.experimental.pallas import tpu_sc as plsc

N = 100000
E = 3200000
DX = 128
DE = 16
DU = 64
B = 128
H = 64

NC = 2
NS = 16
NW = NC * NS

RB = 100
KC = 8
CH = KC * RB
EPW = E // NW
ITERS = EPW // CH
NPAD = 100096
NPT = NPAD // NS


def _scatter_body(src_hbm, attr_hbm, zs_hbm, ones_hbm,
                  sums_out, cnt_out, acc_sh, idx_v, attr_v, ones_v):
  core = lax.axis_index("c")
  sub = lax.axis_index("s")
  w = sub * NC + core
  edge_lo = w * EPW
  row_lo = w * (EPW // RB)

  pltpu.sync_copy(ones_hbm, ones_v)
  pltpu.sync_copy(zs_hbm.at[pl.ds(sub * NPT, NPT)],
                  acc_sh.at[pl.ds(sub * NPT, NPT)])
  plsc.subcore_barrier()

  def attr_chunk(i, carry):
    pltpu.sync_copy(src_hbm.at[pl.ds(row_lo + i * KC, KC)], idx_v)
    pltpu.sync_copy(attr_hbm.at[pl.ds(edge_lo + i * CH, CH)], attr_v)
    for j in range(KC):
      pltpu.sync_copy(attr_v.at[pl.ds(j * RB, RB)],
                      acc_sh.at[idx_v.at[j]], add=True)
    return carry

  lax.fori_loop(0, ITERS, attr_chunk, 0)
  plsc.subcore_barrier()

  pltpu.sync_copy(acc_sh.at[pl.ds(sub * NPT, NPT)],
                  sums_out.at[core, pl.ds(sub * NPT, NPT)])
  plsc.subcore_barrier()

  pltpu.sync_copy(zs_hbm.at[pl.ds(sub * NPT, NPT)],
                  acc_sh.at[pl.ds(sub * NPT, NPT)])
  plsc.subcore_barrier()

  def cnt_chunk(i, carry):
    pltpu.sync_copy(src_hbm.at[pl.ds(row_lo + i * KC, KC)], idx_v)
    for j in range(KC):
      pltpu.sync_copy(ones_v, acc_sh.at[idx_v.at[j]], add=True)
    return carry

  lax.fori_loop(0, ITERS, cnt_chunk, 0)
  plsc.subcore_barrier()

  pltpu.sync_copy(acc_sh.at[pl.ds(sub * NPT, NPT)],
                  cnt_out.at[core, pl.ds(sub * NPT, NPT)])


@functools.cache
def _scatter():
  return pl.kernel(
      _scatter_body,
      out_type=[
          jax.ShapeDtypeStruct((NC, NPAD, DE), jnp.float32),
          jax.ShapeDtypeStruct((NC, NPAD, DE), jnp.float32),
      ],
      mesh=plsc.VectorSubcoreMesh(core_axis_name="c", subcore_axis_name="s",
                                  num_cores=NC, num_subcores=NS),
      compiler_params=pltpu.CompilerParams(use_tc_tiling_on_sc=False),
      scratch_types=[
          pltpu.VMEM_SHARED((NPAD, DE), jnp.float32),
          pltpu.VMEM((KC, RB), jnp.int32),
          pltpu.VMEM((CH, DE), jnp.float32),
          pltpu.VMEM((RB, DE), jnp.float32),
      ],
  )


BN = 1000
GRID = N // BN


def _mlp_body(x_ref, s_ref, c_ref, b_ref, u_ref, w1x_ref, w1e_ref, w1u_ref,
              w2_ref, b1_ref, b2_ref, o_ref):
  f32 = jnp.float32
  hi = jax.lax.Precision.HIGHEST
  s = s_ref[0] + s_ref[1]
  c = c_ref[0, :, 0:1] + c_ref[1, :, 0:1]
  ve = s / jnp.maximum(c, 1.0)
  oh = (b_ref[...] == lax.broadcasted_iota(jnp.int32, (1, B), 1)).astype(f32)
  uproj = jnp.dot(u_ref[...], w1u_ref[...], precision=hi,
                  preferred_element_type=f32)
  acc = (jnp.dot(x_ref[...], w1x_ref[...], precision=hi,
                 preferred_element_type=f32)
         + jnp.dot(ve, w1e_ref[...], precision=hi, preferred_element_type=f32)
         + jnp.dot(oh, uproj, precision=hi, preferred_element_type=f32)
         + b1_ref[...])
  h = jnp.maximum(acc, 0.0)
  o_ref[...] = jnp.maximum(
      jnp.dot(h, w2_ref[...], precision=hi, preferred_element_type=f32)
      + b2_ref[...], 0.0)


def _mlp(x, sums, cnts, batch2, u, w1x, w1e, w1u, w2t, b1r, b2r):
  full = lambda shape: pl.BlockSpec(shape, lambda i: tuple(0 for _ in shape))
  return pl.pallas_call(
      _mlp_body,
      grid=(GRID,),
      in_specs=[
          pl.BlockSpec((BN, DX), lambda i: (i, 0)),
          pl.BlockSpec((NC, BN, DE), lambda i: (0, i, 0)),
          pl.BlockSpec((NC, BN, DE), lambda i: (0, i, 0)),
          pl.BlockSpec((BN, 1), lambda i: (i, 0)),
          full((B, DU)),
          full((DX, H)),
          full((DE, H)),
          full((DU, H)),
          full((H, H)),
          full((1, H)),
          full((1, H)),
      ],
      out_specs=pl.BlockSpec((BN, H), lambda i: (i, 0)),
      out_shape=jax.ShapeDtypeStruct((N, H), jnp.float32),
  )(x, sums, cnts, batch2, u, w1x, w1e, w1u, w2t, b1r, b2r)


def kernel(x, edge_index, edge_attr, u, batch, W1, b1, W2, b2):
  src2 = edge_index[0].reshape(E // RB, RB)
  zs = jnp.zeros((NPAD, DE), jnp.float32)
  ones = jnp.zeros((RB, DE), jnp.float32).at[:, 0].set(1.0)
  sums, cnts = _scatter()(src2, edge_attr, zs, ones)

  W1T = W1.T
  return _mlp(x, sums, cnts, batch.reshape(N, 1), u,
              W1T[:DX], W1T[DX:DX + DE], W1T[DX + DE:], W2.T,
              b1.reshape(1, H), b2.reshape(1, H))
